# baseline (speedup 1.0000x reference)
import functools

import jax
import jax.numpy as jnp
from jax import lax
from jax.experimental import pallas as pl
from jax.experimental.pallas import tpu as pltpu
from jax.experimental.pallas import tpu_sc as plsc

NC = 2; NS = 16; L = 16; NW = 32
TABLE = 256 * 256 * 256
PLANE = 512 * 512
P = 4 * PLANE
PPT = P // NW
C = 4096
NCHUNK = PPT // C
NV = C // L

_mesh = plsc.VectorSubcoreMesh(core_axis_name="c", subcore_axis_name="s")


@functools.partial(
    pl.kernel,
    mesh=_mesh,
    out_type=jax.ShapeDtypeStruct((12 * PLANE,), jnp.float32),
    compiler_params=pltpu.CompilerParams(use_tc_tiling_on_sc=False),
    scratch_types=[
        pltpu.VMEM((C,), jnp.float32),
        pltpu.VMEM((C,), jnp.float32),
        pltpu.VMEM((C,), jnp.float32),
        pltpu.VMEM((C,), jnp.int32),
        pltpu.VMEM((C,), jnp.float32),
        pltpu.VMEM((C,), jnp.float32),
        pltpu.VMEM((C,), jnp.float32),
        pltpu.VMEM((C,), jnp.float32),
        pltpu.VMEM((C,), jnp.float32),
        pltpu.VMEM((C,), jnp.float32),
        pltpu.SemaphoreType.DMA,
    ],
)
def _sc_body(img_hbm, wf_hbm, bf_hbm, out_hbm,
             rv, gv, bv, idx0, w0v, w1v, w2v, b0v, b1v, b2v, sem):
    wid = lax.axis_index("s") * NC + lax.axis_index("c")
    n = wid // 8
    poff0 = (wid % 8) * PPT

    def chunk(j, carry):
        off = poff0 + j * C
        base = n * (3 * PLANE) + off
        pltpu.sync_copy(img_hbm.at[pl.ds(base, C)], rv)
        pltpu.sync_copy(img_hbm.at[pl.ds(base + PLANE, C)], gv)
        pltpu.sync_copy(img_hbm.at[pl.ds(base + 2 * PLANE, C)], bv)

        def mkidx(i, c2):
            s = i * L
            fi = rv[pl.ds(s, L)] * 65536.0 + gv[pl.ds(s, L)] * 256.0 + bv[pl.ds(s, L)]
            idx0[pl.ds(s, L)] = fi.astype(jnp.int32)
            return c2

        lax.fori_loop(0, NV, mkidx, 0)

        def mkidx2(i, c2):
            s = i * L
            idx0[pl.ds(s, L)] = idx0[pl.ds(s, L)] + TABLE
            return c2

        cps0 = [
            pltpu.async_copy(wf_hbm.at[idx0], w0v, sem),
            pltpu.async_copy(bf_hbm.at[idx0], b0v, sem),
        ]
        for cp in cps0:
            cp.wait()
        lax.fori_loop(0, NV, mkidx2, 0)
        cps1 = [
            pltpu.async_copy(wf_hbm.at[idx0], w1v, sem),
            pltpu.async_copy(bf_hbm.at[idx0], b1v, sem),
        ]
        for cp in cps1:
            cp.wait()
        lax.fori_loop(0, NV, mkidx2, 0)
        cps = [
            pltpu.async_copy(wf_hbm.at[idx0], w2v, sem),
            pltpu.async_copy(bf_hbm.at[idx0], b2v, sem),
        ]
        for cp in cps:
            cp.wait()

        def comp(i, c2):
            s = i * L
            for ch, wcol, bcol in ((rv, w0v, b0v), (gv, w1v, b1v), (bv, w2v, b2v)):
                x = ch[pl.ds(s, L)]
                wv = wcol[pl.ds(s, L)]
                bb = bcol[pl.ds(s, L)]
                ch[pl.ds(s, L)] = wv * x + 127.0 * (bb - wv + 1.0)
            return c2

        lax.fori_loop(0, NV, comp, 0)

        pltpu.sync_copy(rv, out_hbm.at[pl.ds(base, C)])
        pltpu.sync_copy(gv, out_hbm.at[pl.ds(base + PLANE, C)])
        pltpu.sync_copy(bv, out_hbm.at[pl.ds(base + 2 * PLANE, C)])
        return carry

    lax.fori_loop(0, NCHUNK, chunk, 0)


def kernel(img, w, b):
    out = _sc_body(img.reshape(-1), w.T.reshape(-1), b.T.reshape(-1))
    return out.reshape(4, 3, 512, 512)


# speedup vs baseline: 5.3146x; 5.3146x over previous
import functools

import jax
import jax.numpy as jnp
from jax import lax
from jax.experimental import pallas as pl
from jax.experimental.pallas import tpu as pltpu
from jax.experimental.pallas import tpu_sc as plsc

NC = 2; NS = 16; L = 16; NW = 32
PLANE = 512 * 512
P = 4 * PLANE
PPT = P // NW
C = 8192
NCHUNK = PPT // C
NV = C // L

_mesh = plsc.VectorSubcoreMesh(core_axis_name="c", subcore_axis_name="s")


@functools.partial(
    pl.kernel,
    mesh=_mesh,
    out_type=jax.ShapeDtypeStruct((12 * PLANE,), jnp.float32),
    compiler_params=pltpu.CompilerParams(use_tc_tiling_on_sc=False),
    scratch_types=[
        pltpu.VMEM((C,), jnp.float32),
        pltpu.VMEM((C,), jnp.float32),
        pltpu.VMEM((C,), jnp.float32),
        pltpu.VMEM((C,), jnp.int32),
        pltpu.VMEM((C,), jnp.float32),
        pltpu.VMEM((C,), jnp.float32),
        pltpu.VMEM((C,), jnp.float32),
        pltpu.VMEM((C,), jnp.float32),
        pltpu.VMEM((C,), jnp.float32),
        pltpu.VMEM((C,), jnp.float32),
        pltpu.SemaphoreType.DMA,
    ],
)
def _sc_body(img_hbm, w0_hbm, w1_hbm, w2_hbm, b0_hbm, b1_hbm, b2_hbm, out_hbm,
             rv, gv, bv, idx0, w0v, w1v, w2v, b0v, b1v, b2v, sem):
    wid = lax.axis_index("s") * NC + lax.axis_index("c")
    n = wid // 8
    poff0 = (wid % 8) * PPT

    def chunk(j, carry):
        off = poff0 + j * C
        base = n * (3 * PLANE) + off
        pltpu.sync_copy(img_hbm.at[pl.ds(base, C)], rv)
        pltpu.sync_copy(img_hbm.at[pl.ds(base + PLANE, C)], gv)
        pltpu.sync_copy(img_hbm.at[pl.ds(base + 2 * PLANE, C)], bv)

        def mkidx(i, c2):
            s = i * L
            fi = rv[pl.ds(s, L)] * 65536.0 + gv[pl.ds(s, L)] * 256.0 + bv[pl.ds(s, L)]
            idx0[pl.ds(s, L)] = fi.astype(jnp.int32)
            return c2

        lax.fori_loop(0, NV, mkidx, 0)

        cps = [
            pltpu.async_copy(w0_hbm.at[idx0], w0v, sem),
            pltpu.async_copy(w1_hbm.at[idx0], w1v, sem),
            pltpu.async_copy(w2_hbm.at[idx0], w2v, sem),
            pltpu.async_copy(b0_hbm.at[idx0], b0v, sem),
            pltpu.async_copy(b1_hbm.at[idx0], b1v, sem),
            pltpu.async_copy(b2_hbm.at[idx0], b2v, sem),
        ]
        for cp in cps:
            cp.wait()

        def comp(i, c2):
            s = i * L
            for ch, wcol, bcol in ((rv, w0v, b0v), (gv, w1v, b1v), (bv, w2v, b2v)):
                x = ch[pl.ds(s, L)]
                wv = wcol[pl.ds(s, L)]
                bb = bcol[pl.ds(s, L)]
                ch[pl.ds(s, L)] = wv * x + 127.0 * (bb - wv + 1.0)
            return c2

        lax.fori_loop(0, NV, comp, 0)

        pltpu.sync_copy(rv, out_hbm.at[pl.ds(base, C)])
        pltpu.sync_copy(gv, out_hbm.at[pl.ds(base + PLANE, C)])
        pltpu.sync_copy(bv, out_hbm.at[pl.ds(base + 2 * PLANE, C)])
        return carry

    lax.fori_loop(0, NCHUNK, chunk, 0)


def kernel(img, w, b):
    out = _sc_body(img.reshape(-1),
                   w[:, 0], w[:, 1], w[:, 2],
                   b[:, 0], b[:, 1], b[:, 2])
    return out.reshape(4, 3, 512, 512)
